# Initial kernel scaffold; baseline (speedup 1.0000x reference)
import jax
import jax.numpy as jnp
from jax.experimental import pallas as pl


def kernel(h_local, hub_features, sWl, sbl, sWr, sbr, satt, sbias, hWl, hbl, hWr, hbr, hatt, hbias, oWl, obl, oWr, obr, oatt, obias, spokes_hubs_edge_index, hubs_batch):
    raise NotImplementedError("write your pallas kernel here")


# speedup vs baseline: 14.6807x; 14.6807x over previous
import functools

import jax
import jax.numpy as jnp
from jax import lax
from jax.experimental import pallas as pl
from jax.experimental.pallas import tpu as pltpu
from jax.experimental.pallas import tpu_sc as plsc

DIM = 512
HEADS = 16
OUT = DIM // HEADS
F32 = jnp.float32

_NC = 2
_NS = 16
_NW = _NC * _NS


def _mm_body(x_ref, w_ref, b_ref, o_ref):
  o_ref[...] = (
      jnp.dot(x_ref[...], w_ref[...], preferred_element_type=F32) + b_ref[...]
  )


def _mm(x, w, b, bm=256):
  m = x.shape[0]
  return pl.pallas_call(
      _mm_body,
      grid=(m // bm,),
      in_specs=[
          pl.BlockSpec((bm, DIM), lambda i: (i, 0)),
          pl.BlockSpec((DIM, DIM), lambda i: (0, 0)),
          pl.BlockSpec((1, DIM), lambda i: (0, 0)),
      ],
      out_specs=pl.BlockSpec((bm, DIM), lambda i: (i, 0)),
      out_shape=jax.ShapeDtypeStruct((m, DIM), F32),
  )(x, w, b.reshape(1, DIM))


_HPG = 4


def _hh_body(xl_ref, xrt_ref, att_ref, bias_ref, o_ref):
  xl = xl_ref[...]
  xrt = xrt_ref[...]
  att = att_ref[...]
  n = xl.shape[0]
  ii = lax.broadcasted_iota(jnp.int32, (n, n), 0)
  jj = lax.broadcasted_iota(jnp.int32, (n, n), 1)
  diag = ii == jj
  outs = []
  for hh in range(_HPG):
    xl_h = xl[:, hh * OUT : (hh + 1) * OUT]
    xrt_h = xrt[hh * OUT : (hh + 1) * OUT, :]
    alpha = jnp.zeros((n, n), F32)
    for k in range(OUT):
      s = xl_h[:, k : k + 1] + xrt_h[k : k + 1, :]
      alpha = alpha + att[0, hh * OUT + k] * jnp.maximum(s, 0.2 * s)
    alpha = jnp.where(diag, -jnp.inf, alpha)
    amax = jnp.max(alpha, axis=0, keepdims=True)
    ex = jnp.exp(alpha - amax)
    denom = jnp.sum(ex, axis=0, keepdims=True)
    a = ex / (denom + 1e-16)
    outs.append(lax.dot_general(
        a, xl_h, (((0,), (0,)), ((), ())), preferred_element_type=F32
    ))
  o_ref[...] = jnp.concatenate(outs, axis=1) + bias_ref[...]


def _hh(xl, xrt, att, bias):
  n = xl.shape[0]
  bw = _HPG * OUT
  return pl.pallas_call(
      _hh_body,
      grid=(HEADS // _HPG,),
      in_specs=[
          pl.BlockSpec((n, bw), lambda g: (0, g)),
          pl.BlockSpec((bw, n), lambda g: (g, 0)),
          pl.BlockSpec((1, bw), lambda g: (0, g)),
          pl.BlockSpec((1, bw), lambda g: (0, g)),
      ],
      out_specs=pl.BlockSpec((n, bw), lambda g: (0, g)),
      out_shape=jax.ShapeDtypeStruct((n, DIM), F32),
  )(xl, xrt, att.reshape(1, DIM), bias.reshape(1, DIM))


def _sc_seg_gatv2(nseg, cap):
  offw = cap + 16
  mesh = plsc.VectorSubcoreMesh(core_axis_name="c", subcore_axis_name="s")

  @functools.partial(
      pl.kernel,
      out_type=jax.ShapeDtypeStruct((nseg, DIM), F32),
      mesh=mesh,
      compiler_params=pltpu.CompilerParams(needs_layout_passes=False),
      scratch_types=[
          pltpu.VMEM((offw,), jnp.int32),
          pltpu.VMEM((32,), jnp.int32),
          pltpu.VMEM((16, DIM), F32),
          pltpu.VMEM((DIM,), F32),
          pltpu.VMEM((32, 16), F32),
          pltpu.VMEM((32, 16), F32),
          pltpu.VMEM((DIM,), F32),
          pltpu.VMEM((DIM,), F32),
          pltpu.VMEM((DIM,), F32),
          pltpu.SemaphoreType.DMA,
      ],
  )
  def k(xl_hbm, xr_hbm, ids_hbm, off_hbm, attf_hbm, bias_hbm, out_hbm,
        off_v, ids_v, rows_v, xr_v, xrt_v, attt_v, bias_v, acc_v, orow_v,
        sem):
    wid = lax.axis_index("s") * _NC + lax.axis_index("c")
    seg0 = pl.multiple_of(wid * cap, 8)
    iota = lax.iota(jnp.int32, 16)
    i32x = iota * 32
    pltpu.sync_copy(off_hbm.at[pl.ds(seg0, offw)], off_v)
    pltpu.sync_copy(bias_hbm, bias_v)
    pltpu.sync_copy(attf_hbm, xr_v)
    for k2 in range(OUT):
      attt_v[k2, :] = plsc.load_gather(xr_v, [i32x + k2])
    nmy = jnp.minimum(cap, nseg - seg0)

    def seg_body(jl, _):
      w16 = plsc.load_gather(off_v, [iota + jl])
      start = w16[0]
      cnt = w16[1] - start
      j = seg0 + jl
      pltpu.sync_copy(xr_hbm.at[j], xr_v)
      for k2 in range(OUT):
        xrt_v[k2, :] = plsc.load_gather(xr_v, [i32x + k2])
      zero16 = jnp.zeros((16,), F32)
      for c in range(32):
        acc_v[pl.ds(c * 16, 16)] = zero16

      def ch_body(ci, carry):
        m16, d16 = carry
        base = start + ci * 16
        albase = pl.multiple_of(jnp.bitwise_and(base, -8), 8)
        pltpu.sync_copy(ids_hbm.at[pl.ds(albase, 32)], ids_v)
        idx16 = plsc.load_gather(ids_v, [iota + (base - albase)])
        pltpu.async_copy(xl_hbm.at[idx16], rows_v, sem).wait()
        rem = jnp.minimum(cnt - ci * 16, 16)

        def e_body(e, ec):
          m16, d16 = ec
          ecol = jnp.full((16,), e, jnp.int32)
          a16 = jnp.zeros((16,), F32)
          for k2 in range(OUT):
            v = plsc.load_gather(rows_v, [ecol, i32x + k2])
            s = v + xrt_v[k2, :]
            a16 = a16 + attt_v[k2, :] * jnp.maximum(s, 0.2 * s)
          mnew = jnp.maximum(m16, a16)
          sc = jnp.exp(m16 - mnew)
          ew = jnp.exp(a16 - mnew)
          d16n = d16 * sc + ew
          for c in range(32):
            h = c // 2
            accc = acc_v[pl.ds(c * 16, 16)]
            acc_v[pl.ds(c * 16, 16)] = (
                accc * sc[h] + ew[h] * rows_v[e, pl.ds(c * 16, 16)]
            )
          return (mnew, d16n)

        return lax.fori_loop(0, rem, e_body, (m16, d16))

      nch = (cnt + 15) >> 4
      m16, d16 = lax.fori_loop(
          0, nch, ch_body,
          (jnp.full((16,), -1e30, F32), jnp.zeros((16,), F32)),
      )
      for c in range(32):
        h = c // 2
        orow_v[pl.ds(c * 16, 16)] = (
            acc_v[pl.ds(c * 16, 16)] / (d16[h] + 1e-16)
            + bias_v[pl.ds(c * 16, 16)]
        )
      pltpu.sync_copy(orow_v, out_hbm.at[j])
      return 0

    lax.fori_loop(0, nmy, seg_body, 0)

  return k


def _pad_to(x, n):
  return jnp.pad(x, (0, n - x.shape[0]))


def _edge_setup(key_ids, sort_by, nseg, cap):
  order = jnp.argsort(sort_by)
  ids = key_ids[order].astype(jnp.int32)
  sorted_by = sort_by[order]
  off = jnp.searchsorted(
      sorted_by, jnp.arange(nseg + 1, dtype=jnp.int32)
  ).astype(jnp.int32)
  ids = _pad_to(ids, ids.shape[0] + 64)
  offw = cap + 16
  off = _pad_to(off, (_NW - 1) * cap + offw)
  return ids, off


def kernel(h_local, hub_features, sWl, sbl, sWr, sbr, satt, sbias,
           hWl, hbl, hWr, hbr, hatt, hbias,
           oWl, obl, oWr, obr, oatt, obias,
           spokes_hubs_edge_index, hubs_batch):
  del hubs_batch
  n_sp = h_local.shape[0]
  n_hub = hub_features.shape[0]
  src = spokes_hubs_edge_index[0]
  dst = spokes_hubs_edge_index[1]

  mpad = ((n_sp + 255) // 256) * 256
  hl_pad = jnp.pad(h_local, ((0, mpad - n_sp), (0, 0)))

  xl1 = _mm(hl_pad, sWl, sbl)
  xr1 = _mm(hub_features, sWr, sbr)
  cap1 = n_hub // _NW
  ids1, off1 = _edge_setup(src, dst, n_hub, cap1)
  sc1 = _sc_seg_gatv2(n_hub, cap1)
  h_glob = sc1(xl1, xr1, ids1, off1, satt.reshape(-1), sbias)

  xl2 = _mm(h_glob, hWl, hbl)
  xr2 = _mm(h_glob, hWr, hbr)
  h_glob = _hh(xl2, xr2.T, hatt, hbias)

  xl3 = _mm(h_glob, oWl, obl)
  xr3 = _mm(hl_pad, oWr, obr)
  cap3 = ((n_sp + _NW - 1) // _NW + 7) // 8 * 8
  ids3, off3 = _edge_setup(dst, src, n_sp, cap3)
  sc3 = _sc_seg_gatv2(n_sp, cap3)
  return sc3(xl3, xr3, ids3, off3, oatt.reshape(-1), obias)


# speedup vs baseline: 15.4910x; 1.0552x over previous
import functools

import jax
import jax.numpy as jnp
from jax import lax
from jax.experimental import pallas as pl
from jax.experimental.pallas import tpu as pltpu
from jax.experimental.pallas import tpu_sc as plsc

DIM = 512
HEADS = 16
OUT = DIM // HEADS
F32 = jnp.float32

_NC = 2
_NS = 16
_NW = _NC * _NS


def _mm_body(x_ref, w_ref, b_ref, o_ref):
  o_ref[...] = (
      jnp.dot(x_ref[...], w_ref[...], preferred_element_type=F32) + b_ref[...]
  )


def _mm(x, w, b, bm=256):
  m = x.shape[0]
  return pl.pallas_call(
      _mm_body,
      grid=(m // bm,),
      in_specs=[
          pl.BlockSpec((bm, DIM), lambda i: (i, 0)),
          pl.BlockSpec((DIM, DIM), lambda i: (0, 0)),
          pl.BlockSpec((1, DIM), lambda i: (0, 0)),
      ],
      out_specs=pl.BlockSpec((bm, DIM), lambda i: (i, 0)),
      out_shape=jax.ShapeDtypeStruct((m, DIM), F32),
  )(x, w, b.reshape(1, DIM))


_HPG = 4


def _hh_body(xl_ref, xrt_ref, att_ref, bias_ref, o_ref):
  xl = xl_ref[...]
  xrt = xrt_ref[...]
  att = att_ref[...]
  n = xl.shape[0]
  ii = lax.broadcasted_iota(jnp.int32, (n, n), 0)
  jj = lax.broadcasted_iota(jnp.int32, (n, n), 1)
  diag = ii == jj
  outs = []
  for hh in range(_HPG):
    xl_h = xl[:, hh * OUT : (hh + 1) * OUT]
    xrt_h = xrt[hh * OUT : (hh + 1) * OUT, :]
    alpha = jnp.zeros((n, n), F32)
    for k in range(OUT):
      s = xl_h[:, k : k + 1] + xrt_h[k : k + 1, :]
      alpha = alpha + att[0, hh * OUT + k] * jnp.maximum(s, 0.2 * s)
    alpha = jnp.where(diag, -jnp.inf, alpha)
    amax = jnp.max(alpha, axis=0, keepdims=True)
    ex = jnp.exp(alpha - amax)
    denom = jnp.sum(ex, axis=0, keepdims=True)
    a = ex / (denom + 1e-16)
    outs.append(lax.dot_general(
        a, xl_h, (((0,), (0,)), ((), ())), preferred_element_type=F32
    ))
  o_ref[...] = jnp.concatenate(outs, axis=1) + bias_ref[...]


def _hh(xl, xrt, att, bias):
  n = xl.shape[0]
  bw = _HPG * OUT
  return pl.pallas_call(
      _hh_body,
      grid=(HEADS // _HPG,),
      in_specs=[
          pl.BlockSpec((n, bw), lambda g: (0, g)),
          pl.BlockSpec((bw, n), lambda g: (g, 0)),
          pl.BlockSpec((1, bw), lambda g: (0, g)),
          pl.BlockSpec((1, bw), lambda g: (0, g)),
      ],
      out_specs=pl.BlockSpec((n, bw), lambda g: (0, g)),
      out_shape=jax.ShapeDtypeStruct((n, DIM), F32),
  )(xl, xrt, att.reshape(1, DIM), bias.reshape(1, DIM))


def _sc_seg_gatv2(nseg, cap):
  offw = cap + 16
  mesh = plsc.VectorSubcoreMesh(core_axis_name="c", subcore_axis_name="s")

  @functools.partial(
      pl.kernel,
      out_type=jax.ShapeDtypeStruct((nseg, DIM), F32),
      mesh=mesh,
      compiler_params=pltpu.CompilerParams(needs_layout_passes=False),
      scratch_types=[
          pltpu.VMEM((offw,), jnp.int32),
          pltpu.VMEM((32,), jnp.int32),
          pltpu.VMEM((16, DIM), F32),
          pltpu.VMEM((8, DIM), F32),
          pltpu.VMEM((32, 16), F32),
          pltpu.VMEM((32, 16), F32),
          pltpu.VMEM((DIM,), F32),
          pltpu.VMEM((DIM,), F32),
          pltpu.VMEM((8, DIM), F32),
          pltpu.SemaphoreType.DMA,
      ],
  )
  def k(xl_hbm, xr_hbm, ids_hbm, off_hbm, attf_hbm, bias_hbm, out_hbm,
        off_v, ids_v, rows_v, xr_v, xrt_v, attt_v, bias_v, acc_v, orow_v,
        sem):
    wid = lax.axis_index("s") * _NC + lax.axis_index("c")
    seg0 = pl.multiple_of(wid * cap, 8)
    iota = lax.iota(jnp.int32, 16)
    i32x = iota * 32
    pltpu.sync_copy(off_hbm.at[pl.ds(seg0, offw)], off_v)
    pltpu.sync_copy(bias_hbm, bias_v)
    pltpu.sync_copy(attf_hbm, acc_v)
    for k2 in range(OUT):
      attt_v[k2, :] = plsc.load_gather(acc_v, [i32x + k2])
    nmy = jnp.minimum(cap, nseg - seg0)

    def blk_body(bi, _):
      j0 = seg0 + bi * 8
      pltpu.sync_copy(xr_hbm.at[pl.ds(j0, 8)], xr_v)
      _seg_block(bi, j0)
      pltpu.sync_copy(orow_v, out_hbm.at[pl.ds(j0, 8)])
      return 0

    def _seg_block(bi, j0):
      lax.fori_loop(0, 8, lambda jl, _: seg_body(bi, jl), 0)

    def seg_body(bi, jl):
      w16 = plsc.load_gather(off_v, [iota + (bi * 8 + jl)])
      start = w16[0]
      cnt = w16[1] - start
      jcol = jnp.full((16,), jl, jnp.int32)
      for k2 in range(OUT):
        xrt_v[k2, :] = plsc.load_gather(xr_v, [jcol, i32x + k2])
      zero16 = jnp.zeros((16,), F32)
      for c in range(32):
        acc_v[pl.ds(c * 16, 16)] = zero16

      def ch_body(ci, carry):
        m16, d16 = carry
        base = start + ci * 16
        albase = pl.multiple_of(jnp.bitwise_and(base, -8), 8)
        pltpu.sync_copy(ids_hbm.at[pl.ds(albase, 32)], ids_v)
        idx16 = plsc.load_gather(ids_v, [iota + (base - albase)])
        pltpu.async_copy(xl_hbm.at[idx16], rows_v, sem).wait()
        rem = jnp.minimum(cnt - ci * 16, 16)

        def e_body(e, ec):
          m16, d16 = ec
          ecol = jnp.full((16,), e, jnp.int32)
          a16 = jnp.zeros((16,), F32)
          for k2 in range(OUT):
            v = plsc.load_gather(rows_v, [ecol, i32x + k2])
            s = v + xrt_v[k2, :]
            a16 = a16 + attt_v[k2, :] * jnp.maximum(s, 0.2 * s)
          mnew = jnp.maximum(m16, a16)
          sc = jnp.exp(m16 - mnew)
          ew = jnp.exp(a16 - mnew)
          d16n = d16 * sc + ew
          for c in range(32):
            h = c // 2
            accc = acc_v[pl.ds(c * 16, 16)]
            acc_v[pl.ds(c * 16, 16)] = (
                accc * sc[h] + ew[h] * rows_v[e, pl.ds(c * 16, 16)]
            )
          return (mnew, d16n)

        return lax.fori_loop(0, rem, e_body, (m16, d16))

      nch = (cnt + 15) >> 4
      m16, d16 = lax.fori_loop(
          0, nch, ch_body,
          (jnp.full((16,), -1e30, F32), jnp.zeros((16,), F32)),
      )
      for c in range(32):
        h = c // 2
        orow_v[jl, pl.ds(c * 16, 16)] = (
            acc_v[pl.ds(c * 16, 16)] / (d16[h] + 1e-16)
            + bias_v[pl.ds(c * 16, 16)]
        )
      return 0

    lax.fori_loop(0, nmy >> 3, blk_body, 0)

  return k


def _pad_to(x, n):
  return jnp.pad(x, (0, n - x.shape[0]))


def _edge_setup(key_ids, sort_by, nseg, cap):
  order = jnp.argsort(sort_by)
  ids = key_ids[order].astype(jnp.int32)
  sorted_by = sort_by[order]
  off = jnp.searchsorted(
      sorted_by, jnp.arange(nseg + 1, dtype=jnp.int32)
  ).astype(jnp.int32)
  ids = _pad_to(ids, ids.shape[0] + 64)
  offw = cap + 16
  off = _pad_to(off, (_NW - 1) * cap + offw)
  return ids, off


def kernel(h_local, hub_features, sWl, sbl, sWr, sbr, satt, sbias,
           hWl, hbl, hWr, hbr, hatt, hbias,
           oWl, obl, oWr, obr, oatt, obias,
           spokes_hubs_edge_index, hubs_batch):
  del hubs_batch
  n_sp = h_local.shape[0]
  n_hub = hub_features.shape[0]
  src = spokes_hubs_edge_index[0]
  dst = spokes_hubs_edge_index[1]

  mpad = ((n_sp + 255) // 256) * 256
  hl_pad = jnp.pad(h_local, ((0, mpad - n_sp), (0, 0)))

  xl1 = _mm(hl_pad, sWl, sbl)
  xr1 = _mm(hub_features, sWr, sbr)
  cap1 = n_hub // _NW
  ids1, off1 = _edge_setup(src, dst, n_hub, cap1)
  sc1 = _sc_seg_gatv2(n_hub, cap1)
  h_glob = sc1(xl1, xr1, ids1, off1, satt.reshape(-1), sbias)

  xl2 = _mm(h_glob, hWl, hbl)
  xr2 = _mm(h_glob, hWr, hbr)
  h_glob = _hh(xl2, xr2.T, hatt, hbias)

  xl3 = _mm(h_glob, oWl, obl)
  xr3 = _mm(hl_pad, oWr, obr)
  cap3 = ((n_sp + _NW - 1) // _NW + 7) // 8 * 8
  ids3, off3 = _edge_setup(dst, src, n_sp, cap3)
  sc3 = _sc_seg_gatv2(n_sp, cap3)
  return sc3(xl3, xr3, ids3, off3, oatt.reshape(-1), obias)


# speedup vs baseline: 16.2439x; 1.0486x over previous
import functools

import jax
import jax.numpy as jnp
from jax import lax
from jax.experimental import pallas as pl
from jax.experimental.pallas import tpu as pltpu
from jax.experimental.pallas import tpu_sc as plsc

DIM = 512
HEADS = 16
OUT = DIM // HEADS
F32 = jnp.float32

_NC = 2
_NS = 16
_NW = _NC * _NS


def _mm_body(x_ref, w_ref, b_ref, o_ref):
  o_ref[...] = (
      jnp.dot(x_ref[...], w_ref[...], preferred_element_type=F32) + b_ref[...]
  )


def _mm(x, w, b, bm=256):
  m = x.shape[0]
  return pl.pallas_call(
      _mm_body,
      grid=(m // bm,),
      in_specs=[
          pl.BlockSpec((bm, DIM), lambda i: (i, 0)),
          pl.BlockSpec((DIM, DIM), lambda i: (0, 0)),
          pl.BlockSpec((1, DIM), lambda i: (0, 0)),
      ],
      out_specs=pl.BlockSpec((bm, DIM), lambda i: (i, 0)),
      out_shape=jax.ShapeDtypeStruct((m, DIM), F32),
  )(x, w, b.reshape(1, DIM))


_HPG = 4


def _hh_body(xl_ref, xrt_ref, att_ref, bias_ref, o_ref):
  xl = xl_ref[...]
  xrt = xrt_ref[...]
  att = att_ref[...]
  n = xl.shape[0]
  ii = lax.broadcasted_iota(jnp.int32, (n, n), 0)
  jj = lax.broadcasted_iota(jnp.int32, (n, n), 1)
  diag = ii == jj
  outs = []
  for hh in range(_HPG):
    xl_h = xl[:, hh * OUT : (hh + 1) * OUT]
    xrt_h = xrt[hh * OUT : (hh + 1) * OUT, :]
    alpha = jnp.zeros((n, n), F32)
    for k in range(OUT):
      s = xl_h[:, k : k + 1] + xrt_h[k : k + 1, :]
      alpha = alpha + att[0, hh * OUT + k] * jnp.maximum(s, 0.2 * s)
    alpha = jnp.where(diag, -jnp.inf, alpha)
    amax = jnp.max(alpha, axis=0, keepdims=True)
    ex = jnp.exp(alpha - amax)
    denom = jnp.sum(ex, axis=0, keepdims=True)
    a = ex / (denom + 1e-16)
    outs.append(lax.dot_general(
        a, xl_h, (((0,), (0,)), ((), ())), preferred_element_type=F32
    ))
  o_ref[...] = jnp.concatenate(outs, axis=1) + bias_ref[...]


def _hh(xl, xrt, att, bias):
  n = xl.shape[0]
  bw = _HPG * OUT
  return pl.pallas_call(
      _hh_body,
      grid=(HEADS // _HPG,),
      in_specs=[
          pl.BlockSpec((n, bw), lambda g: (0, g)),
          pl.BlockSpec((bw, n), lambda g: (g, 0)),
          pl.BlockSpec((1, bw), lambda g: (0, g)),
          pl.BlockSpec((1, bw), lambda g: (0, g)),
      ],
      out_specs=pl.BlockSpec((n, bw), lambda g: (0, g)),
      out_shape=jax.ShapeDtypeStruct((n, DIM), F32),
  )(xl, xrt, att.reshape(1, DIM), bias.reshape(1, DIM))


def _sc_seg_gatv2(nseg, cap, esz, nsrc):
  offw = cap + 16
  del nsrc
  small_src = False
  mesh = plsc.VectorSubcoreMesh(core_axis_name="c", subcore_axis_name="s")

  @functools.partial(
      pl.kernel,
      out_type=jax.ShapeDtypeStruct((nseg, DIM), F32),
      mesh=mesh,
      compiler_params=pltpu.CompilerParams(needs_layout_passes=False),
      scratch_types=[
          pltpu.VMEM((offw,), jnp.int32),
          pltpu.VMEM((esz,), jnp.int32),
          pltpu.VMEM((16, DIM), F32),
          pltpu.VMEM((8, DIM), F32),
          pltpu.VMEM((32, 16), F32),
          pltpu.VMEM((32, 16), F32),
          pltpu.VMEM((DIM,), F32),
          pltpu.VMEM((DIM,), F32),
          pltpu.VMEM((8, DIM), F32),
          pltpu.SemaphoreType.DMA,
      ]
      + ([pltpu.VMEM_SHARED((nsrc, DIM), F32)] if small_src else []),
  )
  def k(xl_hbm, xr_hbm, ids_hbm, off_hbm, attf_hbm, bias_hbm, out_hbm,
        off_v, ids_v, rows_v, xr_v, xrt_v, attt_v, bias_v, acc_v, orow_v,
        sem, *maybe_shared):
    wid = lax.axis_index("s") * _NC + lax.axis_index("c")
    seg0 = pl.multiple_of(wid * cap, 8)
    iota = lax.iota(jnp.int32, 16)
    i32x = iota * 32
    pltpu.sync_copy(off_hbm.at[pl.ds(seg0, offw)], off_v)
    pltpu.sync_copy(bias_hbm, bias_v)
    pltpu.sync_copy(ids_hbm, ids_v)
    if small_src:
      xl_tab = maybe_shared[0]
      sid = pl.multiple_of(lax.axis_index("s") * (nsrc // _NS), 8)
      pltpu.sync_copy(
          xl_hbm.at[pl.ds(sid, nsrc // _NS)], xl_tab.at[pl.ds(sid, nsrc // _NS)]
      )
      plsc.subcore_barrier()
    else:
      xl_tab = xl_hbm
    pltpu.sync_copy(attf_hbm, acc_v)
    for k2 in range(OUT):
      attt_v[k2, :] = plsc.load_gather(acc_v, [i32x + k2])
    nmy = jnp.minimum(cap, nseg - seg0)

    def blk_body(bi, _):
      j0 = seg0 + bi * 8
      pltpu.sync_copy(xr_hbm.at[pl.ds(j0, 8)], xr_v)
      _seg_block(bi, j0)
      pltpu.sync_copy(orow_v, out_hbm.at[pl.ds(j0, 8)])
      return 0

    def _seg_block(bi, j0):
      lax.fori_loop(0, 8, lambda jl, _: seg_body(bi, jl), 0)

    def seg_body(bi, jl):
      w16 = plsc.load_gather(off_v, [iota + (bi * 8 + jl)])
      start = w16[0]
      cnt = w16[1] - start
      jcol = jnp.full((16,), jl, jnp.int32)
      for k2 in range(OUT):
        xrt_v[k2, :] = plsc.load_gather(xr_v, [jcol, i32x + k2])
      zero16 = jnp.zeros((16,), F32)
      for c in range(32):
        acc_v[pl.ds(c * 16, 16)] = zero16

      def ch_body(ci, carry):
        m16, d16 = carry
        base = start + ci * 16
        idx16 = plsc.load_gather(ids_v, [iota + base])
        pltpu.async_copy(xl_tab.at[idx16], rows_v, sem).wait()
        rem = jnp.minimum(cnt - ci * 16, 16)

        def e_body(e, ec):
          m16, d16 = ec
          ecol = jnp.full((16,), e, jnp.int32)
          a16 = jnp.zeros((16,), F32)
          for k2 in range(OUT):
            v = plsc.load_gather(rows_v, [ecol, i32x + k2])
            s = v + xrt_v[k2, :]
            a16 = a16 + attt_v[k2, :] * jnp.maximum(s, 0.2 * s)
          mnew = jnp.maximum(m16, a16)
          sc = jnp.exp(m16 - mnew)
          ew = jnp.exp(a16 - mnew)
          d16n = d16 * sc + ew
          for c in range(32):
            h = c // 2
            accc = acc_v[pl.ds(c * 16, 16)]
            acc_v[pl.ds(c * 16, 16)] = (
                accc * sc[h] + ew[h] * rows_v[e, pl.ds(c * 16, 16)]
            )
          return (mnew, d16n)

        return lax.fori_loop(0, rem, e_body, (m16, d16))

      nch = (cnt + 15) >> 4
      m16, d16 = lax.fori_loop(
          0, nch, ch_body,
          (jnp.full((16,), -1e30, F32), jnp.zeros((16,), F32)),
      )
      for c in range(32):
        h = c // 2
        orow_v[jl, pl.ds(c * 16, 16)] = (
            acc_v[pl.ds(c * 16, 16)] / (d16[h] + 1e-16)
            + bias_v[pl.ds(c * 16, 16)]
        )
      return 0

    lax.fori_loop(0, nmy >> 3, blk_body, 0)

  return k


def _pad_to(x, n):
  return jnp.pad(x, (0, n - x.shape[0]))


def _edge_setup(key_ids, sort_by, nseg, cap):
  order = jnp.argsort(sort_by)
  ids = key_ids[order].astype(jnp.int32)
  sorted_by = sort_by[order]
  off = jnp.searchsorted(
      sorted_by, jnp.arange(nseg + 1, dtype=jnp.int32)
  ).astype(jnp.int32)
  ids = _pad_to(ids, ids.shape[0] + 64)
  offw = cap + 16
  off = _pad_to(off, (_NW - 1) * cap + offw)
  return ids, off


def kernel(h_local, hub_features, sWl, sbl, sWr, sbr, satt, sbias,
           hWl, hbl, hWr, hbr, hatt, hbias,
           oWl, obl, oWr, obr, oatt, obias,
           spokes_hubs_edge_index, hubs_batch):
  del hubs_batch
  n_sp = h_local.shape[0]
  n_hub = hub_features.shape[0]
  src = spokes_hubs_edge_index[0]
  dst = spokes_hubs_edge_index[1]

  mpad = ((n_sp + 255) // 256) * 256
  hl_pad = jnp.pad(h_local, ((0, mpad - n_sp), (0, 0)))

  xl1 = _mm(hl_pad, sWl, sbl)
  xr1 = _mm(hub_features, sWr, sbr)
  cap1 = n_hub // _NW
  ids1, off1 = _edge_setup(src, dst, n_hub, cap1)
  sc1 = _sc_seg_gatv2(n_hub, cap1, ids1.shape[0], mpad)
  h_glob = sc1(xl1, xr1, ids1, off1, satt.reshape(-1), sbias)

  xl2 = _mm(h_glob, hWl, hbl)
  xr2 = _mm(h_glob, hWr, hbr)
  h_glob = _hh(xl2, xr2.T, hatt, hbias)

  xl3 = _mm(h_glob, oWl, obl)
  xr3 = _mm(hl_pad, oWr, obr)
  cap3 = ((n_sp + _NW - 1) // _NW + 7) // 8 * 8
  ids3, off3 = _edge_setup(dst, src, n_sp, cap3)
  sc3 = _sc_seg_gatv2(n_sp, cap3, ids3.shape[0], n_hub)
  return sc3(xl3, xr3, ids3, off3, oatt.reshape(-1), obias)
